# baseline (speedup 1.0000x reference)
import functools

import jax
import jax.numpy as jnp
from jax import lax
from jax.experimental import pallas as pl
from jax.experimental.pallas import tpu as pltpu

D_MODEL = 512
MAX_MEM = 100000
BATCH = 4096
TOP_K = 16
EPS = 1e-12

CB = 2048
RB = 1024
M_PAD = 100352
NCB = M_PAD // CB
NRB = BATCH // RB

NEG_INF = float("-inf")


def _sims_kernel(qk_ref, mk_ref, qid_ref, out_ref):
    j = pl.program_id(1)
    s = lax.dot_general(
        qk_ref[...], mk_ref[...],
        (((1,), (1,)), ((), ())),
        preferred_element_type=jnp.float32,
    )
    col = j * CB + lax.broadcasted_iota(jnp.int32, (RB, CB), 1)
    qid = qid_ref[0, 0, :]
    bad = (col == qid[:, None]) | (col >= MAX_MEM) | (s < 0.0)
    out_ref[...] = jnp.where(bad, NEG_INF, s)


@functools.partial(jax.jit, static_argnames=())
def _compute_sims(query_keys, mem_keys, query_ids):
    qid3 = query_ids.reshape(NRB, 1, RB)
    return pl.pallas_call(
        _sims_kernel,
        grid=(NRB, NCB),
        in_specs=[
            pl.BlockSpec((RB, D_MODEL), lambda i, j: (i, 0)),
            pl.BlockSpec((CB, D_MODEL), lambda i, j: (j, 0)),
            pl.BlockSpec((1, 1, RB), lambda i, j: (i, 0, 0)),
        ],
        out_specs=pl.BlockSpec((RB, CB), lambda i, j: (i, j)),
        out_shape=jax.ShapeDtypeStruct((BATCH, M_PAD), jnp.float32),
    )(query_keys, mem_keys, qid3)


def _l2norm(x):
    n = jnp.linalg.norm(x, ord=2, axis=-1, keepdims=True)
    return x / jnp.maximum(n, EPS)


def kernel(query_representations, memory_raw, query_ids, memory_ids):
    query_keys = _l2norm(query_representations)
    mem_keys = _l2norm(memory_raw)
    sims = _compute_sims(query_keys, mem_keys, query_ids)

    top_sims, top_idx = lax.top_k(sims, TOP_K)
    mask = jnp.isfinite(top_sims)
    sims_out = jnp.where(mask, top_sims, 0.0)
    retrieved = jnp.take(memory_raw, top_idx, axis=0) * mask[..., None].astype(jnp.float32)
    retrieved_ids = jnp.where(mask, jnp.take(memory_ids, top_idx, mode="clip"), -1)
    return retrieved, sims_out, mask, retrieved_ids


# speedup vs baseline: 2.7790x; 2.7790x over previous
import functools

import jax
import jax.numpy as jnp
from jax import lax
from jax.experimental import pallas as pl
from jax.experimental.pallas import tpu as pltpu
from jax.experimental.pallas import tpu_sc as plsc

D_MODEL = 512
MAX_MEM = 100000
BATCH = 4096
TOP_K = 16
EPS = 1e-12

CB = 2048
RB = 1024
M_PAD = 100352
NCB = M_PAD // CB
NRB = BATCH // RB

NEG_INF = float("-inf")

N_WORKERS = 32
ROWS_PER_W = BATCH // N_WORKERS
CHUNKS = (256, 512, 1024, 2048, 4096, 8192, 16384, 32768, 17536, 17536)
CHUNK_MAX = max(CHUNKS)
assert sum(CHUNKS) == M_PAD


def _sims_kernel(qk_ref, mk_ref, qid_ref, out_ref):
    j = pl.program_id(1)
    s = lax.dot_general(
        qk_ref[...], mk_ref[...],
        (((1,), (1,)), ((), ())),
        preferred_element_type=jnp.float32,
    )
    col = j * CB + lax.broadcasted_iota(jnp.int32, (RB, CB), 1)
    qid = qid_ref[0, 0, :]
    bad = (col == qid[:, None]) | (col >= MAX_MEM) | (s < 0.0)
    out_ref[...] = jnp.where(bad, NEG_INF, s)


def _compute_sims(query_keys, mem_keys, query_ids):
    qid3 = query_ids.reshape(NRB, 1, RB)
    return pl.pallas_call(
        _sims_kernel,
        grid=(NRB, NCB),
        in_specs=[
            pl.BlockSpec((RB, D_MODEL), lambda i, j: (i, 0)),
            pl.BlockSpec((CB, D_MODEL), lambda i, j: (j, 0)),
            pl.BlockSpec((1, 1, RB), lambda i, j: (i, 0, 0)),
        ],
        out_specs=pl.BlockSpec((RB, CB), lambda i, j: (i, j)),
        out_shape=jax.ShapeDtypeStruct((BATCH, M_PAD), jnp.float32),
    )(query_keys, mem_keys, qid3)


def _merge_topk(top_v, top_i, cand_v, cand_i):
    cv, ci = plsc.sort_key_val(cand_v, cand_i, descending=True)
    sel = cv > top_v
    mv = jnp.where(sel, cv, top_v)
    mi = jnp.where(sel, ci, top_i)
    sv, si = plsc.sort_key_val(mv, mi, descending=False)
    return sv, si


GROUP = 8


def _topk_body(sims_ref, mem_ref, ret_ref, sims_out_ref, ids_out_ref,
               buf0, buf1, cand, idxv, curv, rows, sims_acc, ids_acc,
               dsem0, dsem1, gsem, osem):
    wid = lax.axis_index("s") * 2 + lax.axis_index("c")
    lane = lax.broadcasted_iota(jnp.int32, (TOP_K,), 0)
    bufs = (buf0, buf1)
    dsems = (dsem0, dsem1)

    def row_body(r, _):
        row_off = (wid * ROWS_PER_W + r) * M_PAD

        pltpu.async_copy(
            sims_ref.at[pl.ds(row_off, CHUNKS[0])],
            buf0.at[pl.ds(0, CHUNKS[0])], dsem0)
        pltpu.async_copy(
            sims_ref.at[pl.ds(row_off + CHUNKS[0], CHUNKS[1])],
            buf1.at[pl.ds(0, CHUNKS[1])], dsem1)

        top_v = jnp.full((TOP_K,), NEG_INF, jnp.float32)
        top_i = jnp.zeros((TOP_K,), jnp.int32)
        thresh = jnp.full((TOP_K,), NEG_INF, jnp.float32)
        t_s = NEG_INF

        base = 0
        for c, size in enumerate(CHUNKS):
            buf = bufs[c % 2]
            pltpu.make_async_copy(
                sims_ref.at[pl.ds(row_off + base, size)],
                buf.at[pl.ds(0, size)], dsems[c % 2]).wait()

            curv[...] = jnp.zeros((TOP_K,), jnp.int32)

            def filt(g, _):
                goff = g * (GROUP * TOP_K)
                mx = buf[pl.ds(goff, TOP_K)]
                for k in range(1, GROUP):
                    mx = jnp.maximum(mx, buf[pl.ds(goff + k * TOP_K, TOP_K)])

                @pl.when(jnp.max(mx) > t_s)
                def _hot():
                    cur = curv[...]
                    for k in range(GROUP):
                        v = buf[pl.ds(goff + k * TOP_K, TOP_K)]
                        m = v > thresh
                        pos = plsc.cumsum(jnp.where(m, 1, 0))
                        tgt = cur + pos - 1
                        plsc.store_scatter(
                            cand, [tgt], goff + k * TOP_K + lane, mask=m)
                        cur = cur + plsc.all_reduce_population_count(m)
                    curv[...] = cur
                return 0

            lax.fori_loop(0, size // (GROUP * TOP_K), filt, 0)

            cur = curv[...]
            n = jnp.max(cur)
            n_splat = cur

            def merge(b, carry):
                tv, ti = carry
                lidx = cand[pl.ds(b * TOP_K, TOP_K)]
                ok = (b * TOP_K + lane) < n_splat
                vals = plsc.load_gather(buf, [lidx], mask=ok)
                vals = jnp.where(ok, vals, NEG_INF)
                return _merge_topk(tv, ti, vals, lidx + base)

            nb = (n + TOP_K - 1) // TOP_K
            top_v, top_i = lax.fori_loop(0, nb, merge, (top_v, top_i))
            t_s = jnp.min(top_v)
            thresh = jnp.broadcast_to(t_s, (TOP_K,))

            if c + 2 < len(CHUNKS):
                nbase = base + size + CHUNKS[c + 1]
                pltpu.async_copy(
                    sims_ref.at[pl.ds(row_off + nbase, CHUNKS[c + 2])],
                    buf.at[pl.ds(0, CHUNKS[c + 2])], dsems[c % 2])
            base += size

        dv = lax.rev(top_v, (0,))
        di = lax.rev(top_i, (0,))
        valid = dv >= 0.0
        sv = jnp.where(valid, dv, 0.0)
        si = jnp.where(valid, di, -1)
        sims_acc[pl.ds(r * TOP_K, TOP_K)] = sv
        ids_acc[pl.ds(r * TOP_K, TOP_K)] = si
        idxv[...] = jnp.where(valid, di, 0)

        grow = wid * ROWS_PER_W + r
        pltpu.async_copy(mem_ref.at[idxv], rows, gsem).wait()
        for i in range(TOP_K):
            @pl.when(si[i] < 0)
            def _zero():
                for j2 in range(D_MODEL // TOP_K):
                    rows[i, pl.ds(j2 * TOP_K, TOP_K)] = jnp.zeros(
                        (TOP_K,), jnp.float32)
        pltpu.async_copy(rows, ret_ref.at[grow], osem).wait()
        return 0

    lax.fori_loop(0, ROWS_PER_W, row_body, 0)
    pltpu.sync_copy(sims_acc, sims_out_ref.at[pl.ds(wid * ROWS_PER_W * TOP_K,
                                                    ROWS_PER_W * TOP_K)])
    pltpu.sync_copy(ids_acc, ids_out_ref.at[pl.ds(wid * ROWS_PER_W * TOP_K,
                                                  ROWS_PER_W * TOP_K)])


def _topk_sc(sims_flat, memory_raw):
    mesh = plsc.VectorSubcoreMesh(core_axis_name="c", subcore_axis_name="s")
    f = pl.kernel(
        _topk_body,
        out_type=(
            jax.ShapeDtypeStruct((BATCH, TOP_K, D_MODEL), jnp.float32),
            jax.ShapeDtypeStruct((BATCH * TOP_K,), jnp.float32),
            jax.ShapeDtypeStruct((BATCH * TOP_K,), jnp.int32),
        ),
        mesh=mesh,
        compiler_params=pltpu.CompilerParams(needs_layout_passes=False),
        scratch_types=[
            pltpu.VMEM((CHUNK_MAX,), jnp.float32),
            pltpu.VMEM((CHUNK_MAX,), jnp.float32),
            pltpu.VMEM((CHUNK_MAX,), jnp.int32),
            pltpu.VMEM((TOP_K,), jnp.int32),
            pltpu.VMEM((TOP_K,), jnp.int32),
            pltpu.VMEM((TOP_K, D_MODEL), jnp.float32),
            pltpu.VMEM((ROWS_PER_W * TOP_K,), jnp.float32),
            pltpu.VMEM((ROWS_PER_W * TOP_K,), jnp.int32),
            pltpu.SemaphoreType.DMA,
            pltpu.SemaphoreType.DMA,
            pltpu.SemaphoreType.DMA,
            pltpu.SemaphoreType.DMA,
        ],
    )
    return f(sims_flat, memory_raw)


def _l2norm(x):
    n = jnp.linalg.norm(x, ord=2, axis=-1, keepdims=True)
    return x / jnp.maximum(n, EPS)


def kernel(query_representations, memory_raw, query_ids, memory_ids):
    query_keys = _l2norm(query_representations)
    mem_keys = _l2norm(memory_raw)
    sims = _compute_sims(query_keys, mem_keys, query_ids)
    retrieved, sims_flat, ids_flat = _topk_sc(sims.reshape(-1), memory_raw)
    sims_out = sims_flat.reshape(BATCH, TOP_K)
    retrieved_ids = ids_flat.reshape(BATCH, TOP_K)
    mask = retrieved_ids >= 0
    return retrieved, sims_out, mask, retrieved_ids


# speedup vs baseline: 3.3558x; 1.2075x over previous
import functools

import jax
import jax.numpy as jnp
from jax import lax
from jax.experimental import pallas as pl
from jax.experimental.pallas import tpu as pltpu
from jax.experimental.pallas import tpu_sc as plsc

D_MODEL = 512
MAX_MEM = 100000
BATCH = 4096
TOP_K = 16
EPS = 1e-12

CB = 2048
RB = 1024
M_PAD = 100352
NCB = M_PAD // CB
NRB = BATCH // RB

NEG_INF = float("-inf")

N_WORKERS = 32
ROWS_PER_W = BATCH // N_WORKERS
CHUNKS = (256, 512, 1024, 2048, 4096, 8192, 16384, 32768, 17536, 17536)
CHUNK_MAX = max(CHUNKS)
assert sum(CHUNKS) == M_PAD


def _sims_kernel(qk_ref, mk_ref, qid_ref, out_ref):
    j = pl.program_id(1)
    s = lax.dot_general(
        qk_ref[...], mk_ref[...],
        (((1,), (1,)), ((), ())),
        preferred_element_type=jnp.float32,
    )
    col = j * CB + lax.broadcasted_iota(jnp.int32, (RB, CB), 1)
    qid = qid_ref[0, 0, :]
    bad = (col == qid[:, None]) | (col >= MAX_MEM) | (s < 0.0)
    out_ref[...] = jnp.where(bad, NEG_INF, s)


def _compute_sims(query_keys, mem_keys, query_ids):
    qid3 = query_ids.reshape(NRB, 1, RB)
    return pl.pallas_call(
        _sims_kernel,
        grid=(NRB, NCB),
        in_specs=[
            pl.BlockSpec((RB, D_MODEL), lambda i, j: (i, 0)),
            pl.BlockSpec((CB, D_MODEL), lambda i, j: (j, 0)),
            pl.BlockSpec((1, 1, RB), lambda i, j: (i, 0, 0)),
        ],
        out_specs=pl.BlockSpec((RB, CB), lambda i, j: (i, j)),
        out_shape=jax.ShapeDtypeStruct((BATCH, M_PAD), jnp.float32),
    )(query_keys, mem_keys, qid3)


def _merge_topk(top_v, top_i, cand_v, cand_i):
    cv, ci = plsc.sort_key_val(cand_v, cand_i, descending=True)
    sel = cv > top_v
    mv = jnp.where(sel, cv, top_v)
    mi = jnp.where(sel, ci, top_i)
    sv, si = plsc.sort_key_val(mv, mi, descending=False)
    return sv, si


GROUP = 8


def _topk_body(sims_ref, mem_ref, ret_ref, sims_out_ref, ids_out_ref,
               buf0, buf1, cand, idxv, curv, rows, sims_acc, ids_acc,
               dsem0, dsem1, gsem, osem):
    wid = lax.axis_index("s") * 2 + lax.axis_index("c")
    lane = lax.broadcasted_iota(jnp.int32, (TOP_K,), 0)
    bufs = (buf0, buf1)
    dsems = (dsem0, dsem1)

    def row_body(r, _):
        row = wid * ROWS_PER_W + r

        pltpu.async_copy(
            sims_ref.at[row, pl.ds(0, CHUNKS[0])],
            buf0.at[pl.ds(0, CHUNKS[0])], dsem0)
        pltpu.async_copy(
            sims_ref.at[row, pl.ds(CHUNKS[0], CHUNKS[1])],
            buf1.at[pl.ds(0, CHUNKS[1])], dsem1)

        top_v = jnp.full((TOP_K,), NEG_INF, jnp.float32)
        top_i = jnp.zeros((TOP_K,), jnp.int32)
        thresh = jnp.full((TOP_K,), NEG_INF, jnp.float32)
        t_s = NEG_INF

        base = 0
        for c, size in enumerate(CHUNKS):
            buf = bufs[c % 2]
            pltpu.make_async_copy(
                sims_ref.at[row, pl.ds(base, size)],
                buf.at[pl.ds(0, size)], dsems[c % 2]).wait()

            curv[...] = jnp.zeros((TOP_K,), jnp.int32)

            def filt(g, _):
                goff = g * (GROUP * TOP_K)
                mx = buf[pl.ds(goff, TOP_K)]
                for k in range(1, GROUP):
                    mx = jnp.maximum(mx, buf[pl.ds(goff + k * TOP_K, TOP_K)])

                @pl.when(jnp.max(mx) > t_s)
                def _hot():
                    cur = curv[...]
                    for k in range(GROUP):
                        v = buf[pl.ds(goff + k * TOP_K, TOP_K)]
                        m = v > thresh
                        pos = plsc.cumsum(jnp.where(m, 1, 0))
                        tgt = cur + pos - 1
                        plsc.store_scatter(
                            cand, [tgt], goff + k * TOP_K + lane, mask=m)
                        cur = cur + plsc.all_reduce_population_count(m)
                    curv[...] = cur
                return 0

            lax.fori_loop(0, size // (GROUP * TOP_K), filt, 0)

            cur = curv[...]
            n = jnp.max(cur)
            n_splat = cur

            def merge(b, carry):
                tv, ti = carry
                lidx = cand[pl.ds(b * TOP_K, TOP_K)]
                ok = (b * TOP_K + lane) < n_splat
                vals = plsc.load_gather(buf, [lidx], mask=ok)
                vals = jnp.where(ok, vals, NEG_INF)
                return _merge_topk(tv, ti, vals, lidx + base)

            nb = (n + TOP_K - 1) // TOP_K
            top_v, top_i = lax.fori_loop(0, nb, merge, (top_v, top_i))
            t_s = jnp.min(top_v)
            thresh = jnp.broadcast_to(t_s, (TOP_K,))

            if c + 2 < len(CHUNKS):
                nbase = base + size + CHUNKS[c + 1]
                pltpu.async_copy(
                    sims_ref.at[row, pl.ds(nbase, CHUNKS[c + 2])],
                    buf.at[pl.ds(0, CHUNKS[c + 2])], dsems[c % 2])
            base += size

        dv = lax.rev(top_v, (0,))
        di = lax.rev(top_i, (0,))
        valid = dv >= 0.0
        sv = jnp.where(valid, dv, 0.0)
        si = jnp.where(valid, di, -1)
        sims_acc[pl.ds(r * TOP_K, TOP_K)] = sv
        ids_acc[pl.ds(r * TOP_K, TOP_K)] = si
        idxv[...] = jnp.where(valid, di, 0)

        pltpu.async_copy(mem_ref.at[idxv], rows, gsem).wait()
        for i in range(TOP_K):
            @pl.when(si[i] < 0)
            def _zero():
                for j2 in range(D_MODEL // TOP_K):
                    rows[i, pl.ds(j2 * TOP_K, TOP_K)] = jnp.zeros(
                        (TOP_K,), jnp.float32)
        pltpu.async_copy(rows, ret_ref.at[row], osem).wait()
        return 0

    lax.fori_loop(0, ROWS_PER_W, row_body, 0)
    pltpu.sync_copy(sims_acc, sims_out_ref.at[pl.ds(wid * ROWS_PER_W * TOP_K,
                                                    ROWS_PER_W * TOP_K)])
    pltpu.sync_copy(ids_acc, ids_out_ref.at[pl.ds(wid * ROWS_PER_W * TOP_K,
                                                  ROWS_PER_W * TOP_K)])


def _topk_sc(sims, memory_raw):
    mesh = plsc.VectorSubcoreMesh(core_axis_name="c", subcore_axis_name="s")
    f = pl.kernel(
        _topk_body,
        out_type=(
            jax.ShapeDtypeStruct((BATCH, TOP_K, D_MODEL), jnp.float32),
            jax.ShapeDtypeStruct((BATCH * TOP_K,), jnp.float32),
            jax.ShapeDtypeStruct((BATCH * TOP_K,), jnp.int32),
        ),
        mesh=mesh,
        compiler_params=pltpu.CompilerParams(needs_layout_passes=False),
        scratch_types=[
            pltpu.VMEM((CHUNK_MAX,), jnp.float32),
            pltpu.VMEM((CHUNK_MAX,), jnp.float32),
            pltpu.VMEM((CHUNK_MAX,), jnp.int32),
            pltpu.VMEM((TOP_K,), jnp.int32),
            pltpu.VMEM((TOP_K,), jnp.int32),
            pltpu.VMEM((TOP_K, D_MODEL), jnp.float32),
            pltpu.VMEM((ROWS_PER_W * TOP_K,), jnp.float32),
            pltpu.VMEM((ROWS_PER_W * TOP_K,), jnp.int32),
            pltpu.SemaphoreType.DMA,
            pltpu.SemaphoreType.DMA,
            pltpu.SemaphoreType.DMA,
            pltpu.SemaphoreType.DMA,
        ],
    )
    return f(sims, memory_raw)


def _l2norm(x):
    n = jnp.linalg.norm(x, ord=2, axis=-1, keepdims=True)
    return x / jnp.maximum(n, EPS)


def kernel(query_representations, memory_raw, query_ids, memory_ids):
    query_keys = _l2norm(query_representations)
    mem_keys = _l2norm(memory_raw)
    sims = _compute_sims(query_keys, mem_keys, query_ids)
    retrieved, sims_flat, ids_flat = _topk_sc(sims, memory_raw)
    sims_out = sims_flat.reshape(BATCH, TOP_K)
    retrieved_ids = ids_flat.reshape(BATCH, TOP_K)
    mask = retrieved_ids >= 0
    return retrieved, sims_out, mask, retrieved_ids
